# baseline (speedup 1.0000x reference)
import jax
import jax.numpy as jnp
from jax import lax
from jax.experimental import pallas as pl
from jax.experimental.pallas import tpu as pltpu

_NUM_CODES = 512
_LATENT_DIM = 256
_BT = 1024


def _vq_tc_kernel(z_ref, cb_ref, zq_ref, idx_ref, minsum_ref):
    i = pl.program_id(0)
    zb = z_ref[...]
    cb = cb_ref[...]
    s = jnp.dot(zb, cb.T, preferred_element_type=jnp.float32)
    zz = jnp.sum(zb * zb, axis=1, keepdims=True)
    ee = jnp.sum(cb * cb, axis=1)[None, :]
    d = (zz - 2.0 * s) + ee
    dmin = jnp.min(d, axis=1, keepdims=True)
    iota_f = lax.broadcasted_iota(jnp.int32, d.shape, 1).astype(jnp.float32)
    idxcol = jnp.min(
        jnp.where(d == dmin, iota_f, float(_NUM_CODES)), axis=1, keepdims=True
    )
    oh = jnp.where(iota_f == idxcol, 1.0, 0.0)

    k_col = lax.broadcasted_iota(jnp.int32, (_NUM_CODES, 1), 0)
    lo = (k_col & 255).astype(jnp.float32)
    hi = (k_col >> 8).astype(jnp.float32)
    cb_ext = jnp.concatenate([cb, lo, hi], axis=1)

    r = lax.dot_general(
        cb_ext, oh, (((0,), (1,)), ((), ())),
        preferred_element_type=jnp.float32,
    )
    zq_ref[0, :, :] = r[:_LATENT_DIM, :]
    idx_row = r[_LATENT_DIM, :] + 256.0 * r[_LATENT_DIM + 1, :]
    idx_ref[0, 0, :] = idx_row.astype(jnp.int32)

    @pl.when(i == 0)
    def _():
        minsum_ref[0, 0] = 0.0

    minsum_ref[0, 0] += jnp.sum(dmin)


def kernel(z, codebook):
    B, C, H, W = z.shape
    z_flat = jnp.transpose(z, (0, 2, 3, 1)).reshape(-1, C)
    n_tok = B * H * W
    grid = n_tok // _BT
    zq_t, idx3, minsum = pl.pallas_call(
        _vq_tc_kernel,
        grid=(grid,),
        in_specs=[
            pl.BlockSpec((_BT, _LATENT_DIM), lambda i: (i, 0)),
            pl.BlockSpec((_NUM_CODES, _LATENT_DIM), lambda i: (0, 0)),
        ],
        out_specs=[
            pl.BlockSpec((1, _LATENT_DIM, _BT), lambda i: (i, 0, 0)),
            pl.BlockSpec((1, 1, _BT), lambda i: (i, 0, 0)),
            pl.BlockSpec(memory_space=pltpu.SMEM),
        ],
        out_shape=[
            jax.ShapeDtypeStruct((grid, _LATENT_DIM, _BT), jnp.float32),
            jax.ShapeDtypeStruct((grid, 1, _BT), jnp.int32),
            jax.ShapeDtypeStruct((1, 1), jnp.float32),
        ],
    )(z_flat, codebook)
    z_q = zq_t.reshape(B, C, H, W)
    loss = minsum[0, 0] * (1.25 / (B * C * H * W))
    return z_q, idx3.reshape(n_tok), loss


# speedup vs baseline: 1.5604x; 1.5604x over previous
import jax
import jax.numpy as jnp
from jax import lax
from jax.experimental import pallas as pl
from jax.experimental.pallas import tpu as pltpu

_NUM_CODES = 512
_LATENT_DIM = 256
_BT = 1024


def _vq_tc_kernel(z_ref, cb_ref, zq_ref, idx_ref, minsum_ref):
    i = pl.program_id(0)
    zb = z_ref[...]
    cb = cb_ref[...]
    s = jnp.dot(zb, cb.T, preferred_element_type=jnp.float32)
    zz = jnp.sum(zb * zb, axis=1, keepdims=True)
    ee = jnp.sum(cb * cb, axis=1)[None, :]
    d = (zz - 2.0 * s) + ee
    dmin = jnp.min(d, axis=1, keepdims=True)
    iota_f = lax.broadcasted_iota(jnp.int32, d.shape, 1).astype(jnp.float32)
    idxcol = jnp.min(
        jnp.where(d == dmin, iota_f, float(_NUM_CODES)), axis=1, keepdims=True
    )
    oh = jnp.where(iota_f == idxcol, 1.0, 0.0)

    k_col = lax.broadcasted_iota(jnp.int32, (_NUM_CODES, 1), 0)
    lo = (k_col & 255).astype(jnp.float32)
    hi = (k_col >> 8).astype(jnp.float32)
    lohi = jnp.concatenate([lo, hi], axis=1)

    zq_ref[...] = lax.dot_general(
        oh, cb, (((1,), (0,)), ((), ())),
        preferred_element_type=jnp.float32,
    )
    r_ix = lax.dot_general(
        lohi, oh, (((0,), (1,)), ((), ())),
        preferred_element_type=jnp.float32,
    )
    idx_row = r_ix[0, :] + 256.0 * r_ix[1, :]
    idx_ref[0, 0, :] = idx_row.astype(jnp.int32)

    @pl.when(i == 0)
    def _():
        minsum_ref[0, 0] = 0.0

    minsum_ref[0, 0] += jnp.sum(dmin)


def kernel(z, codebook):
    B, C, H, W = z.shape
    z_flat = jnp.transpose(z, (0, 2, 3, 1)).reshape(-1, C)
    n_tok = B * H * W
    grid = n_tok // _BT
    zq_t, idx3, minsum = pl.pallas_call(
        _vq_tc_kernel,
        grid=(grid,),
        in_specs=[
            pl.BlockSpec((_BT, _LATENT_DIM), lambda i: (i, 0)),
            pl.BlockSpec((_NUM_CODES, _LATENT_DIM), lambda i: (0, 0)),
        ],
        out_specs=[
            pl.BlockSpec((_BT, _LATENT_DIM), lambda i: (i, 0)),
            pl.BlockSpec((1, 1, _BT), lambda i: (i, 0, 0)),
            pl.BlockSpec(memory_space=pltpu.SMEM),
        ],
        out_shape=[
            jax.ShapeDtypeStruct((n_tok, _LATENT_DIM), jnp.float32),
            jax.ShapeDtypeStruct((grid, 1, _BT), jnp.int32),
            jax.ShapeDtypeStruct((1, 1), jnp.float32),
        ],
    )(z_flat, codebook)
    z_q = jnp.transpose(zq_t.reshape(B, H, W, C), (0, 3, 1, 2))
    loss = minsum[0, 0] * (1.25 / (B * C * H * W))
    return z_q, idx3.reshape(n_tok), loss
